# baseline (speedup 1.0000x reference)
import functools

import jax
import jax.numpy as jnp
from jax import lax
from jax.experimental import pallas as pl
from jax.experimental.pallas import tpu as pltpu
from jax.experimental.pallas import tpu_sc as plsc

N_NODES = 10000
DIM = 128
N_EDGES = 320000

NPAD = 10240
NC, NS = 2, 16
CHUNK = 128
CH_PER_TILE = 80
E_PAD = NC * NS * CH_PER_TILE * CHUNK
MCHUNK = 64
MCH_PER_TILE = E_PAD // (NC * NS * MCHUNK)
ROWS_PER_TILE = NPAD // NS
MSG_PHASES = 4
CH_C0 = 320
CH_C1 = 0
PH_C0 = CH_C0 // MSG_PHASES
PH_C1 = CH_C1 // MSG_PHASES
PH_MAX = max(PH_C0, PH_C1)
RBLK = 1024

_mesh = plsc.VectorSubcoreMesh(core_axis_name="c", subcore_axis_name="s")


NW = NC * NS
EDGES_PER_TILE = E_PAD // NW
DVECS = EDGES_PER_TILE // 16
DUNROLL = 4


@functools.partial(
    pl.kernel,
    mesh=_mesh,
    out_type=jax.ShapeDtypeStruct((NW * NPAD,), jnp.float32),
    scratch_types=[
        pltpu.VMEM((EDGES_PER_TILE,), jnp.int32),
        pltpu.VMEM((NPAD,), jnp.float32),
    ],
    compiler_params=pltpu.CompilerParams(needs_layout_passes=False),
)
def _deg_kernel(dst_hbm, zeros_hbm, out_hbm, dst_v, deg_l):
    c = lax.axis_index("c")
    s = lax.axis_index("s")
    w = c * NS + s
    pltpu.sync_copy(dst_hbm.at[pl.ds(w * EDGES_PER_TILE, EDGES_PER_TILE)], dst_v)
    pltpu.sync_copy(zeros_hbm, deg_l)

    def body(i, carry):
        for u in range(DUNROLL):
            v = dst_v[pl.ds((i * DUNROLL + u) * 16, 16)]
            counts, lastm = plsc.scan_count(v)
            plsc.addupdate_scatter(
                deg_l, [v], counts.astype(jnp.float32), mask=lastm)
        return carry

    lax.fori_loop(0, DVECS // DUNROLL, body, 0)
    pltpu.sync_copy(deg_l, out_hbm.at[pl.ds(w * NPAD, NPAD)])


@functools.partial(
    pl.kernel,
    mesh=_mesh,
    out_type=jax.ShapeDtypeStruct((NC, NPAD, DIM), jnp.float32),
    scratch_types=[
        pltpu.VMEM((PH_MAX, MCHUNK), jnp.int32),
        pltpu.VMEM((PH_MAX, MCHUNK), jnp.int32),
        pltpu.VMEM((MCHUNK, DIM), jnp.float32),
        pltpu.VMEM((MCHUNK, DIM), jnp.float32),
        pltpu.VMEM_SHARED((NPAD, DIM), jnp.float32),
        pltpu.SemaphoreType.DMA,
        pltpu.SemaphoreType.DMA,
    ],
)
def _scatter_kernel(g_hbm, src_hbm, dst_hbm, zeros_hbm, out_hbm,
                    src_v, dst_v, buf_a, buf_b, acc_sh, sem_a, sem_b):
    c = lax.axis_index("c")
    s = lax.axis_index("s")
    ch_c = jnp.where(c == 0, CH_C0, CH_C1)
    ph_c = jnp.where(c == 0, PH_C0, PH_C1)
    tile_base = jnp.where(c == 0, s * CH_C0, NS * CH_C0 + s * CH_C1)
    pltpu.sync_copy(zeros_hbm, acc_sh.at[pl.ds(s * ROWS_PER_TILE, ROWS_PER_TILE)])
    plsc.subcore_barrier()

    def body(it, carry):
        j = 2 * it
        pltpu.make_async_copy(g_hbm.at[src_v.at[j]], buf_a, sem_a).wait()
        pltpu.make_async_copy(g_hbm.at[src_v.at[j + 1]], buf_b, sem_b).start()
        pltpu.sync_copy(buf_a, acc_sh.at[dst_v.at[j]], add=True)
        pltpu.make_async_copy(g_hbm.at[src_v.at[j + 1]], buf_b, sem_b).wait()

        @pl.when(j + 2 < ph_c)
        def _():
            pltpu.make_async_copy(g_hbm.at[src_v.at[j + 2]], buf_a, sem_a).start()

        pltpu.sync_copy(buf_b, acc_sh.at[dst_v.at[j + 1]], add=True)
        return carry

    for ph in range(MSG_PHASES):
        @pl.when(ph_c > 0)
        def _():
            base = tile_base + ph * ph_c
            pltpu.sync_copy(src_hbm.at[pl.ds(base, PH_MAX)], src_v)
            pltpu.sync_copy(dst_hbm.at[pl.ds(base, PH_MAX)], dst_v)
            pltpu.make_async_copy(g_hbm.at[src_v.at[0]], buf_a, sem_a).start()
            lax.fori_loop(0, ph_c // 2, body, 0)
    plsc.subcore_barrier()
    pltpu.sync_copy(
        acc_sh.at[pl.ds(s * ROWS_PER_TILE, ROWS_PER_TILE)],
        out_hbm.at[c, pl.ds(s * ROWS_PER_TILE, ROWS_PER_TILE)],
    )


def _mm_body(deg_ref, x_ref, w_ref, g_ref):
    d = jnp.sum(deg_ref[...], axis=1, keepdims=True) + 1.0
    dinv = lax.rsqrt(d)
    g_ref[...] = jnp.dot(x_ref[...], w_ref[...],
                         preferred_element_type=jnp.float32) * dinv


def _comb_body(acc_ref, g_ref, deg_ref, b_ref, o_ref):
    d = jnp.sum(deg_ref[...], axis=1, keepdims=True) + 1.0
    dinv = lax.rsqrt(d)
    acc = acc_ref[...]
    o_ref[...] = (acc[0] + acc[1] + g_ref[...]) * dinv + b_ref[...]


def kernel(x, edge_index, W, b):
    src = edge_index[0].astype(jnp.int32)
    dst = edge_index[1].astype(jnp.int32)
    pad = jnp.full((E_PAD - N_EDGES,), N_NODES, jnp.int32)
    src_flat = jnp.concatenate([src, pad])
    dst_flat = jnp.concatenate([dst, pad])
    srcp = src_flat.reshape(E_PAD // MCHUNK, MCHUNK)
    dstp_m = dst_flat.reshape(E_PAD // MCHUNK, MCHUNK)
    x_pad = jnp.zeros((NPAD, DIM), jnp.float32).at[:N_NODES].set(x)

    zeros_acc = jnp.zeros((ROWS_PER_TILE, DIM), jnp.float32)
    zeros_deg = jnp.zeros((NPAD,), jnp.float32)

    degp = _deg_kernel(dst_flat, zeros_deg)
    deg_t = degp.reshape(NW, NPAD).T

    ngrid = NPAD // RBLK
    g = pl.pallas_call(
        _mm_body,
        grid=(ngrid,),
        in_specs=[
            pl.BlockSpec((RBLK, NW), lambda i: (i, 0)),
            pl.BlockSpec((RBLK, DIM), lambda i: (i, 0)),
            pl.BlockSpec((DIM, DIM), lambda i: (0, 0)),
        ],
        out_specs=pl.BlockSpec((RBLK, DIM), lambda i: (i, 0)),
        out_shape=jax.ShapeDtypeStruct((NPAD, DIM), jnp.float32),
    )(deg_t, x_pad, W)

    accp = _scatter_kernel(g, srcp, dstp_m, zeros_acc)

    outp = pl.pallas_call(
        _comb_body,
        grid=(ngrid,),
        in_specs=[
            pl.BlockSpec((NC, RBLK, DIM), lambda i: (0, i, 0)),
            pl.BlockSpec((RBLK, DIM), lambda i: (i, 0)),
            pl.BlockSpec((RBLK, NW), lambda i: (i, 0)),
            pl.BlockSpec((1, DIM), lambda i: (0, 0)),
        ],
        out_specs=pl.BlockSpec((RBLK, DIM), lambda i: (i, 0)),
        out_shape=jax.ShapeDtypeStruct((NPAD, DIM), jnp.float32),
    )(accp, g, deg_t, b.reshape(1, DIM))

    return outp[:N_NODES]


# speedup vs baseline: 1.6904x; 1.6904x over previous
import functools

import jax
import jax.numpy as jnp
from jax import lax
from jax.experimental import pallas as pl
from jax.experimental.pallas import tpu as pltpu
from jax.experimental.pallas import tpu_sc as plsc

N_NODES = 10000
DIM = 128
N_EDGES = 320000

NPAD = 10240
NC, NS = 2, 16
CHUNK = 128
CH_PER_TILE = 80
E_PAD = NC * NS * CH_PER_TILE * CHUNK
MCHUNK = 64
MCH_PER_TILE = E_PAD // (NC * NS * MCHUNK)
ROWS_PER_TILE = NPAD // NS
MSG_PHASES = 4
CH_C0 = 288
CH_C1 = 32
PH_C0 = CH_C0 // MSG_PHASES
PH_C1 = CH_C1 // MSG_PHASES
PH_MAX = max(PH_C0, PH_C1)
RBLK = 1024

_mesh = plsc.VectorSubcoreMesh(core_axis_name="c", subcore_axis_name="s")


NW = NC * NS
EDGES_PER_TILE = E_PAD // NW
DVECS = EDGES_PER_TILE // 16
DUNROLL = 4


@functools.partial(
    pl.kernel,
    mesh=_mesh,
    out_type=jax.ShapeDtypeStruct((NW * NPAD,), jnp.float32),
    scratch_types=[
        pltpu.VMEM((EDGES_PER_TILE,), jnp.int32),
        pltpu.VMEM((NPAD,), jnp.float32),
    ],
    compiler_params=pltpu.CompilerParams(needs_layout_passes=False),
)
def _deg_kernel(dst_hbm, zeros_hbm, out_hbm, dst_v, deg_l):
    c = lax.axis_index("c")
    s = lax.axis_index("s")
    w = c * NS + s
    pltpu.sync_copy(dst_hbm.at[pl.ds(w * EDGES_PER_TILE, EDGES_PER_TILE)], dst_v)
    pltpu.sync_copy(zeros_hbm, deg_l)

    def body(i, carry):
        for u in range(DUNROLL):
            v = dst_v[pl.ds((i * DUNROLL + u) * 16, 16)]
            counts, lastm = plsc.scan_count(v)
            plsc.addupdate_scatter(
                deg_l, [v], counts.astype(jnp.float32), mask=lastm)
        return carry

    lax.fori_loop(0, DVECS // DUNROLL, body, 0)
    pltpu.sync_copy(deg_l, out_hbm.at[pl.ds(w * NPAD, NPAD)])


@functools.partial(
    pl.kernel,
    mesh=_mesh,
    out_type=jax.ShapeDtypeStruct((NC, NPAD, DIM), jnp.float32),
    scratch_types=[
        pltpu.VMEM((PH_MAX, MCHUNK), jnp.int32),
        pltpu.VMEM((PH_MAX, MCHUNK), jnp.int32),
        pltpu.VMEM((MCHUNK, DIM), jnp.float32),
        pltpu.VMEM((MCHUNK, DIM), jnp.float32),
        pltpu.VMEM_SHARED((NPAD, DIM), jnp.float32),
        pltpu.SemaphoreType.DMA,
        pltpu.SemaphoreType.DMA,
    ],
)
def _scatter_kernel(g_hbm, src_hbm, dst_hbm, zeros_hbm, out_hbm,
                    src_v, dst_v, buf_a, buf_b, acc_sh, sem_a, sem_b):
    c = lax.axis_index("c")
    s = lax.axis_index("s")
    ch_c = jnp.where(c == 0, CH_C0, CH_C1)
    ph_c = jnp.where(c == 0, PH_C0, PH_C1)
    tile_base = jnp.where(c == 0, s * CH_C0, NS * CH_C0 + s * CH_C1)
    pltpu.sync_copy(zeros_hbm, acc_sh.at[pl.ds(s * ROWS_PER_TILE, ROWS_PER_TILE)])
    plsc.subcore_barrier()

    def body(it, carry):
        j = 2 * it
        pltpu.make_async_copy(g_hbm.at[src_v.at[j]], buf_a, sem_a).wait()
        pltpu.make_async_copy(g_hbm.at[src_v.at[j + 1]], buf_b, sem_b).start()
        pltpu.sync_copy(buf_a, acc_sh.at[dst_v.at[j]], add=True)
        pltpu.make_async_copy(g_hbm.at[src_v.at[j + 1]], buf_b, sem_b).wait()

        @pl.when(j + 2 < ph_c)
        def _():
            pltpu.make_async_copy(g_hbm.at[src_v.at[j + 2]], buf_a, sem_a).start()

        pltpu.sync_copy(buf_b, acc_sh.at[dst_v.at[j + 1]], add=True)
        return carry

    for ph in range(MSG_PHASES):
        @pl.when(ph_c > 0)
        def _():
            base = tile_base + ph * ph_c
            pltpu.sync_copy(src_hbm.at[pl.ds(base, PH_MAX)], src_v)
            pltpu.sync_copy(dst_hbm.at[pl.ds(base, PH_MAX)], dst_v)
            pltpu.make_async_copy(g_hbm.at[src_v.at[0]], buf_a, sem_a).start()
            lax.fori_loop(0, ph_c // 2, body, 0)
    plsc.subcore_barrier()
    pltpu.sync_copy(
        acc_sh.at[pl.ds(s * ROWS_PER_TILE, ROWS_PER_TILE)],
        out_hbm.at[c, pl.ds(s * ROWS_PER_TILE, ROWS_PER_TILE)],
    )


def _mm_body(deg_ref, x_ref, w_ref, g_ref):
    d = jnp.sum(deg_ref[...], axis=1, keepdims=True) + 1.0
    dinv = lax.rsqrt(d)
    g_ref[...] = jnp.dot(x_ref[...], w_ref[...],
                         preferred_element_type=jnp.float32) * dinv


def _comb_body(acc_ref, g_ref, deg_ref, b_ref, o_ref):
    d = jnp.sum(deg_ref[...], axis=1, keepdims=True) + 1.0
    dinv = lax.rsqrt(d)
    acc = acc_ref[...]
    o_ref[...] = (acc[0] + acc[1] + g_ref[...]) * dinv + b_ref[...]


def kernel(x, edge_index, W, b):
    src = edge_index[0].astype(jnp.int32)
    dst = edge_index[1].astype(jnp.int32)
    pad = jnp.full((E_PAD - N_EDGES,), N_NODES, jnp.int32)
    src_flat = jnp.concatenate([src, pad])
    dst_flat = jnp.concatenate([dst, pad])
    srcp = src_flat.reshape(E_PAD // MCHUNK, MCHUNK)
    dstp_m = dst_flat.reshape(E_PAD // MCHUNK, MCHUNK)
    x_pad = jnp.zeros((NPAD, DIM), jnp.float32).at[:N_NODES].set(x)

    zeros_acc = jnp.zeros((ROWS_PER_TILE, DIM), jnp.float32)
    zeros_deg = jnp.zeros((NPAD,), jnp.float32)

    degp = _deg_kernel(dst_flat, zeros_deg)
    deg_t = degp.reshape(NW, NPAD).T

    ngrid = NPAD // RBLK
    g = pl.pallas_call(
        _mm_body,
        grid=(ngrid,),
        in_specs=[
            pl.BlockSpec((RBLK, NW), lambda i: (i, 0)),
            pl.BlockSpec((RBLK, DIM), lambda i: (i, 0)),
            pl.BlockSpec((DIM, DIM), lambda i: (0, 0)),
        ],
        out_specs=pl.BlockSpec((RBLK, DIM), lambda i: (i, 0)),
        out_shape=jax.ShapeDtypeStruct((NPAD, DIM), jnp.float32),
    )(deg_t, x_pad, W)

    accp = _scatter_kernel(g, srcp, dstp_m, zeros_acc)

    outp = pl.pallas_call(
        _comb_body,
        grid=(ngrid,),
        in_specs=[
            pl.BlockSpec((NC, RBLK, DIM), lambda i: (0, i, 0)),
            pl.BlockSpec((RBLK, DIM), lambda i: (i, 0)),
            pl.BlockSpec((RBLK, NW), lambda i: (i, 0)),
            pl.BlockSpec((1, DIM), lambda i: (0, 0)),
        ],
        out_specs=pl.BlockSpec((RBLK, DIM), lambda i: (i, 0)),
        out_shape=jax.ShapeDtypeStruct((NPAD, DIM), jnp.float32),
    )(accp, g, deg_t, b.reshape(1, DIM))

    return outp[:N_NODES]
